# baseline (speedup 1.0000x reference)
import functools

import jax
import jax.numpy as jnp
from jax import lax
from jax.experimental import pallas as pl
from jax.experimental.pallas import tpu as pltpu
from jax.experimental.pallas import tpu_sc as plsc

IN_SIZE = 128
OUT_SIZE = 64
BATCH = 16384

_info = plsc.get_sparse_core_info()
_NC, _NS = _info.num_cores, _info.num_subcores
_NW = _NC * _NS
_BPW = BATCH // _NW
_CH = 128
_NCH = _BPW // _CH


@functools.partial(
    pl.kernel,
    mesh=plsc.VectorSubcoreMesh(core_axis_name="c", subcore_axis_name="s"),
    out_type=jax.ShapeDtypeStruct((BATCH, OUT_SIZE), jnp.float32),
    scratch_types=[
        pltpu.VMEM((_NCH, _CH), jnp.int32),
        pltpu.VMEM((_BPW, OUT_SIZE), jnp.float32),
        pltpu.SemaphoreType.DMA,
    ],
    compiler_params=pltpu.CompilerParams(use_tc_tiling_on_sc=False),
)
def _sc_gather(table_hbm, idx_hbm, out_hbm, idx_v, rows_v, sem):
    wid = lax.axis_index("s") * _NC + lax.axis_index("c")
    base = wid * _BPW
    pltpu.sync_copy(idx_hbm.at[wid], idx_v)
    copies = [
        pltpu.async_copy(
            table_hbm.at[idx_v.at[j]],
            rows_v.at[pl.ds(j * _CH, _CH)],
            sem,
        )
        for j in range(_NCH)
    ]
    for c in copies:
        c.wait()
    pltpu.sync_copy(rows_v, out_hbm.at[pl.ds(base, _BPW)])


def _fc_mul(x_ref, w_ref, wi_ref, o_ref):
    fc = lax.dot_general(
        x_ref[...], w_ref[...],
        (((1,), (1,)), ((), ())),
        preferred_element_type=jnp.float32,
    )
    o_ref[...] = wi_ref[...] * fc


_BLK = 1024


def kernel(x, id, fc_weight, emb_weight):
    idx3 = id.astype(jnp.int32).reshape(_NW, _NCH, _CH)
    w_i = _sc_gather(emb_weight, idx3)
    out = pl.pallas_call(
        _fc_mul,
        grid=(BATCH // _BLK,),
        in_specs=[
            pl.BlockSpec((_BLK, IN_SIZE), lambda i: (i, 0)),
            pl.BlockSpec((OUT_SIZE, IN_SIZE), lambda i: (0, 0)),
            pl.BlockSpec((_BLK, OUT_SIZE), lambda i: (i, 0)),
        ],
        out_specs=pl.BlockSpec((_BLK, OUT_SIZE), lambda i: (i, 0)),
        out_shape=jax.ShapeDtypeStruct((BATCH, OUT_SIZE), jnp.float32),
    )(x, fc_weight, w_i)
    return out
